# baseline (speedup 1.0000x reference)
import jax
import jax.numpy as jnp
from jax import lax
from jax.experimental import pallas as pl
from jax.experimental.pallas import tpu as pltpu
from jax.experimental.pallas import tpu_sc as plsc

NC, NS, L = 2, 16, 16
NW = NC * NS
ROWS, COLS = 128, 32768
RPW = ROWS // NW
K = 8
GROUP = 16
GW = GROUP * L
NGRP = COLS // GW
NG8 = NGRP // L
CAP = 2048
QCAP = 32
NEG = -100000.0
IMAX = 2**31 - 1


def _body(scores_hbm, kofs_hbm, masked_hbm, vals_hbm, idx_hbm,
          row0_v, row1_v, neg_v_buf, gmax_v, cmax_v, cvals_v, cidx_v,
          pstage_v, kofs_v, pack_f, pack_i,
          sin0, sin1, sneg0, sneg1, sneg2, sneg3, sscat, spack):
    rowbufs = [row0_v, row1_v]
    sneg = [sneg0, sneg1, sneg2, sneg3]
    wid = lax.axis_index("s") * NC + lax.axis_index("c")
    lanes = lax.broadcasted_iota(jnp.int32, (L,), 0)
    ninf = jnp.float32(-jnp.inf)
    ninf_v = jnp.full((L,), ninf, jnp.float32)
    neg_vec = jnp.full((L,), NEG, jnp.float32)
    zero_i = jnp.zeros((L,), jnp.int32)
    row_base = wid * RPW

    in_h = [None] * RPW
    in_h[0] = pltpu.async_copy(scores_hbm.at[row_base], rowbufs[0], sin0)
    in_h_sem = [sin0, sin1]

    pltpu.sync_copy(kofs_hbm, kofs_v)
    kofs = jnp.max(kofs_v[...])

    def negfill(j, _c):
        for u in range(8):
            neg_v_buf[pl.ds((j * 8 + u) * L, L)] = neg_vec
        return 0
    lax.fori_loop(0, COLS // (8 * L), negfill, 0)
    neg_h = [pltpu.async_copy(neg_v_buf, masked_hbm.at[row_base + r], sneg[r])
             for r in range(RPW)]

    tvpack = ninf_v
    tipack = zero_i
    q_hist = [None] * RPW

    for r in range(RPW):
        buf = rowbufs[r % 2]
        row = row_base + r
        in_h[r].wait()
        if r + 1 < RPW:
            in_h[r + 1] = pltpu.async_copy(
                scores_hbm.at[row + 1], rowbufs[(r + 1) % 2],
                in_h_sem[(r + 1) % 2])

        def grp1(gp, lm, buf=buf):
            gms = []
            for h in range(2):
                g = gp * 2 + h
                acc = [ninf_v] * 4
                for j in range(GROUP):
                    acc[j % 4] = jnp.maximum(
                        acc[j % 4], buf[pl.ds((g * GROUP + j) * L, L)])
                gm = jnp.maximum(jnp.maximum(acc[0], acc[1]),
                                 jnp.maximum(acc[2], acc[3]))
                gmax_v[pl.ds(g * L, L)] = gm
                gms.append(gm)
            return jnp.maximum(lm, jnp.maximum(gms[0], gms[1]))
        lm = lax.fori_loop(0, NGRP // 2, grp1, ninf_v)

        t = ninf
        for _i in range(K):
            t = jnp.max(lm)
            lm = jnp.where(lm == t, ninf_v, lm)

        gcol = lanes * GROUP

        def qscan(g8, _c):
            acc = [ninf_v] * 4
            for c in range(GROUP):
                acc[c % 4] = jnp.maximum(
                    acc[c % 4],
                    plsc.load_gather(gmax_v, [g8 * (L * GROUP) + gcol + c]))
            cm = jnp.maximum(jnp.maximum(acc[0], acc[1]),
                             jnp.maximum(acc[2], acc[3]))
            cmax_v[pl.ds(g8 * L, L)] = cm
            return 0
        lax.fori_loop(0, NG8, qscan, 0)

        def coll_g8(g8, offv, buf=buf, t=t):
            qb = cmax_v[pl.ds(g8 * L, L)] >= t
            cnt = jnp.sum(qb.astype(jnp.int32))

            def one(_i, carry):
                qb, offv = carry
                gl = jnp.max(plsc.all_reduce_ffs(qb))
                g = g8 * L + gl
                for j in range(GROUP):
                    base = (g * GROUP + j) * L
                    v = buf[pl.ds(base, L)]
                    m = v >= t
                    mi = m.astype(jnp.int32)
                    pos = jnp.minimum(offv + plsc.cumsum(mi) - mi, CAP)
                    plsc.store_scatter(cvals_v, [pos], v, mask=m)
                    plsc.store_scatter(cidx_v, [pos], lanes + base, mask=m)
                    offv = offv + plsc.all_reduce_population_count(m)
                qb = jnp.where(lanes == gl, False, qb)
                return qb, offv
            _qb, offv = lax.fori_loop(0, cnt, one, (qb, offv))
            return offv
        offv = lax.fori_loop(0, NG8, coll_g8, zero_i)
        used = jnp.minimum(jnp.max(offv), CAP)
        nv = (used + L - 1) // L
        cvals_v[pl.ds(used, L)] = ninf_v

        lane_base = (r % 2) * K

        def single_rounds(carry, lane_base=lane_base):
            tv, ti = carry
            cv = cvals_v[pl.ds(0, L)]
            ci = cidx_v[pl.ds(0, L)]
            v8 = ninf
            for i in range(K):
                mx = jnp.max(cv)
                mix = jnp.min(jnp.where(cv == mx, ci, IMAX))
                cv = jnp.where(ci == mix, ninf_v, cv)
                tv = jnp.where(lanes == lane_base + i, mx, tv)
                ti = jnp.where(lanes == lane_base + i, mix, ti)
                v8 = mx
            return tv, ti, v8

        def multi_rounds(carry, nv=nv, lane_base=lane_base):
            tv, ti = carry

            def round_fn(i, c2):
                tv, ti, _v8 = c2

                def amax(jv, m):
                    return jnp.maximum(m, cvals_v[pl.ds(jv * L, L)])
                mx = jnp.max(lax.fori_loop(0, nv, amax, ninf_v))

                def amin(jv, mi):
                    cv = cvals_v[pl.ds(jv * L, L)]
                    ci = cidx_v[pl.ds(jv * L, L)]
                    return jnp.minimum(mi, jnp.where(cv == mx, ci, IMAX))
                mix = jnp.min(lax.fori_loop(0, nv, amin,
                                            jnp.full((L,), IMAX, jnp.int32)))

                def rem(jv, _c):
                    cv = cvals_v[pl.ds(jv * L, L)]
                    ci = cidx_v[pl.ds(jv * L, L)]
                    cvals_v[pl.ds(jv * L, L)] = jnp.where(ci == mix, ninf_v,
                                                          cv)
                    return 0
                lax.fori_loop(0, nv, rem, 0)
                tv = jnp.where(lanes == lane_base + i, mx, tv)
                ti = jnp.where(lanes == lane_base + i, mix, ti)
                return tv, ti, mx
            return lax.fori_loop(0, K, round_fn, (tv, ti, ninf))

        tvpack, tipack, v8 = lax.cond(nv == 1, single_rounds, multi_rounds,
                                      (tvpack, tipack))
        if r % 2 == 1:
            pack_f[pl.ds((r // 2) * L, L)] = tvpack + kofs
            pack_i[pl.ds((r // 2) * L, L)] = tipack
            tvpack = ninf_v
            tipack = zero_i

        neg_h[r].wait()
        if r >= 1:
            def drain(_j, _c):
                pltpu.make_async_copy(
                    scores_hbm.at[row_base].at[pl.ds(0, GW)],
                    pstage_v.at[pl.ds(0, GW)], sscat).wait()
                return 0
            lax.fori_loop(0, q_hist[r - 1], drain, 0)

        def patch_g8(g8, q, buf=buf, row=row, v8=v8):
            pb = cmax_v[pl.ds(g8 * L, L)] >= v8
            pcnt = jnp.sum(pb.astype(jnp.int32))

            def onep(_i, carry):
                pb, q = carry
                gl = jnp.max(plsc.all_reduce_ffs(pb))
                g = g8 * L + gl
                q_c = jnp.minimum(q, QCAP - 1)
                for j in range(GROUP):
                    v = buf[pl.ds((g * GROUP + j) * L, L)]
                    pstage_v[pl.ds(q_c * GW + j * L, L)] = jnp.where(
                        v >= v8, v + kofs, neg_vec)

                @pl.when(q < QCAP)
                def _():
                    pltpu.async_copy(
                        pstage_v.at[pl.ds(q_c * GW, GW)],
                        masked_hbm.at[row].at[pl.ds(g * GW, GW)], sscat)
                pb = jnp.where(lanes == gl, False, pb)
                return pb, q + 1
            _pb, q = lax.fori_loop(0, pcnt, onep, (pb, q))
            return q
        q = lax.fori_loop(0, NG8, patch_g8, jnp.int32(0))
        q_hist[r] = jnp.minimum(q, QCAP)

    def drain_last(_j, _c):
        pltpu.make_async_copy(
            scores_hbm.at[row_base].at[pl.ds(0, GW)],
            pstage_v.at[pl.ds(0, GW)], sscat).wait()
        return 0
    lax.fori_loop(0, q_hist[RPW - 1], drain_last, 0)

    pltpu.async_copy(pack_f, vals_hbm.at[pl.ds(row_base * K, RPW * K)],
                     spack).wait()
    pltpu.async_copy(pack_i, idx_hbm.at[pl.ds(row_base * K, RPW * K)],
                     spack).wait()


def kernel(scores, k):
    kofs = jnp.full((L,), 1.0, jnp.float32) * (
        jnp.asarray(k, jnp.int32) - K).astype(jnp.float32)
    mesh = plsc.VectorSubcoreMesh(core_axis_name="c", subcore_axis_name="s",
                                  num_cores=NC, num_subcores=NS)
    f = pl.kernel(
        _body,
        out_type=[
            jax.ShapeDtypeStruct((ROWS, COLS), jnp.float32),
            jax.ShapeDtypeStruct((ROWS * K,), jnp.float32),
            jax.ShapeDtypeStruct((ROWS * K,), jnp.int32),
        ],
        mesh=mesh,
        compiler_params=pltpu.CompilerParams(needs_layout_passes=False),
        scratch_types=[
            pltpu.VMEM((COLS,), jnp.float32),
            pltpu.VMEM((COLS,), jnp.float32),
            pltpu.VMEM((COLS,), jnp.float32),
            pltpu.VMEM((NGRP * L,), jnp.float32),
            pltpu.VMEM((NGRP,), jnp.float32),
            pltpu.VMEM((CAP + L,), jnp.float32),
            pltpu.VMEM((CAP + L,), jnp.int32),
            pltpu.VMEM((QCAP * GW,), jnp.float32),
            pltpu.VMEM((L,), jnp.float32),
            pltpu.VMEM((RPW * K,), jnp.float32),
            pltpu.VMEM((RPW * K,), jnp.int32),
            pltpu.SemaphoreType.DMA,
            pltpu.SemaphoreType.DMA,
            pltpu.SemaphoreType.DMA,
            pltpu.SemaphoreType.DMA,
            pltpu.SemaphoreType.DMA,
            pltpu.SemaphoreType.DMA,
            pltpu.SemaphoreType.DMA,
            pltpu.SemaphoreType.DMA,
        ],
    )
    masked, vals, idx = f(scores, kofs)
    return masked, vals.reshape(ROWS, K), idx.reshape(ROWS, K)


# speedup vs baseline: 1.0103x; 1.0103x over previous
import jax
import jax.numpy as jnp
from jax import lax
from jax.experimental import pallas as pl
from jax.experimental.pallas import tpu as pltpu
from jax.experimental.pallas import tpu_sc as plsc

NC, NS, L = 2, 16, 16
NW = NC * NS
ROWS, COLS = 128, 32768
RPW = ROWS // NW
K = 8
GROUP = 16
GW = GROUP * L
NGRP = COLS // GW
NG8 = NGRP // L
CAP = 2048
QCAP = 32
NEG = -100000.0
IMAX = 2**31 - 1


def _body(scores_hbm, kofs_hbm, masked_hbm, vals_hbm, idx_hbm,
          row0_v, row1_v, neg_sh, gmax_v, cmax_v, cvals_v, cidx_v,
          pstage_v, kofs_v, pack_f, pack_i,
          sin0, sin1, sneg0, sneg1, sneg2, sneg3, sscat, spack):
    rowbufs = [row0_v, row1_v]
    sneg = [sneg0, sneg1, sneg2, sneg3]
    wid = lax.axis_index("s") * NC + lax.axis_index("c")
    lanes = lax.broadcasted_iota(jnp.int32, (L,), 0)
    ninf = jnp.float32(-jnp.inf)
    ninf_v = jnp.full((L,), ninf, jnp.float32)
    neg_vec = jnp.full((L,), NEG, jnp.float32)
    zero_i = jnp.zeros((L,), jnp.int32)
    row_base = wid * RPW

    in_h = [None] * RPW
    in_h[0] = pltpu.async_copy(scores_hbm.at[row_base], rowbufs[0], sin0)
    in_h_sem = [sin0, sin1]

    pltpu.sync_copy(kofs_hbm, kofs_v)
    kofs = jnp.max(kofs_v[...])

    sid = lax.axis_index("s")
    chunk = COLS // NS

    def negfill(j, _c):
        for u in range(8):
            pstage_v[pl.ds((j * 8 + u) * L, L)] = neg_vec
        return 0
    lax.fori_loop(0, chunk // (8 * L), negfill, 0)
    pltpu.sync_copy(pstage_v.at[pl.ds(0, chunk)],
                    neg_sh.at[pl.ds(sid * chunk, chunk)])
    plsc.subcore_barrier()
    neg_h = [pltpu.async_copy(neg_sh, masked_hbm.at[row_base + r], sneg[r])
             for r in range(RPW)]

    tvpack = ninf_v
    tipack = zero_i
    q_hist = [None] * RPW

    for r in range(RPW):
        buf = rowbufs[r % 2]
        row = row_base + r
        in_h[r].wait()
        if r + 1 < RPW:
            in_h[r + 1] = pltpu.async_copy(
                scores_hbm.at[row + 1], rowbufs[(r + 1) % 2],
                in_h_sem[(r + 1) % 2])

        def grp1(g, lm, buf=buf):
            acc = [ninf_v] * 4
            for j in range(GROUP):
                acc[j % 4] = jnp.maximum(
                    acc[j % 4], buf[pl.ds((g * GROUP + j) * L, L)])
            gm = jnp.maximum(jnp.maximum(acc[0], acc[1]),
                             jnp.maximum(acc[2], acc[3]))
            gmax_v[pl.ds(g * L, L)] = gm
            return jnp.maximum(lm, gm)
        lm = lax.fori_loop(0, NGRP, grp1, ninf_v)

        t = ninf
        for _i in range(K):
            t = jnp.max(lm)
            lm = jnp.where(lm == t, ninf_v, lm)

        gcol = lanes * GROUP

        def qscan(g8, _c):
            acc = [ninf_v] * 4
            for c in range(GROUP):
                acc[c % 4] = jnp.maximum(
                    acc[c % 4],
                    plsc.load_gather(gmax_v, [g8 * (L * GROUP) + gcol + c]))
            cm = jnp.maximum(jnp.maximum(acc[0], acc[1]),
                             jnp.maximum(acc[2], acc[3]))
            cmax_v[pl.ds(g8 * L, L)] = cm
            return 0
        lax.fori_loop(0, NG8, qscan, 0)

        def coll_g8(g8, offv, buf=buf, t=t):
            qb = cmax_v[pl.ds(g8 * L, L)] >= t
            cnt = jnp.sum(qb.astype(jnp.int32))

            def one(_i, carry):
                qb, offv = carry
                gl = jnp.max(plsc.all_reduce_ffs(qb))
                g = g8 * L + gl
                for j in range(GROUP):
                    base = (g * GROUP + j) * L
                    v = buf[pl.ds(base, L)]
                    m = v >= t
                    mi = m.astype(jnp.int32)
                    pos = jnp.minimum(offv + plsc.cumsum(mi) - mi, CAP)
                    plsc.store_scatter(cvals_v, [pos], v, mask=m)
                    plsc.store_scatter(cidx_v, [pos], lanes + base, mask=m)
                    offv = offv + plsc.all_reduce_population_count(m)
                qb = jnp.where(lanes == gl, False, qb)
                return qb, offv
            _qb, offv = lax.fori_loop(0, cnt, one, (qb, offv))
            return offv
        offv = lax.fori_loop(0, NG8, coll_g8, zero_i)
        used = jnp.minimum(jnp.max(offv), CAP)
        nv = (used + L - 1) // L
        cvals_v[pl.ds(used, L)] = ninf_v

        lane_base = (r % 2) * K

        def single_rounds(carry, lane_base=lane_base):
            tv, ti = carry
            cv = cvals_v[pl.ds(0, L)]
            ci = cidx_v[pl.ds(0, L)]
            v8 = ninf
            for i in range(K):
                mx = jnp.max(cv)
                mix = jnp.min(jnp.where(cv == mx, ci, IMAX))
                cv = jnp.where(ci == mix, ninf_v, cv)
                tv = jnp.where(lanes == lane_base + i, mx, tv)
                ti = jnp.where(lanes == lane_base + i, mix, ti)
                v8 = mx
            return tv, ti, v8

        def multi_rounds(carry, nv=nv, lane_base=lane_base):
            tv, ti = carry

            def round_fn(i, c2):
                tv, ti, _v8 = c2

                def amax(jv, m):
                    return jnp.maximum(m, cvals_v[pl.ds(jv * L, L)])
                mx = jnp.max(lax.fori_loop(0, nv, amax, ninf_v))

                def amin(jv, mi):
                    cv = cvals_v[pl.ds(jv * L, L)]
                    ci = cidx_v[pl.ds(jv * L, L)]
                    return jnp.minimum(mi, jnp.where(cv == mx, ci, IMAX))
                mix = jnp.min(lax.fori_loop(0, nv, amin,
                                            jnp.full((L,), IMAX, jnp.int32)))

                def rem(jv, _c):
                    cv = cvals_v[pl.ds(jv * L, L)]
                    ci = cidx_v[pl.ds(jv * L, L)]
                    cvals_v[pl.ds(jv * L, L)] = jnp.where(ci == mix, ninf_v,
                                                          cv)
                    return 0
                lax.fori_loop(0, nv, rem, 0)
                tv = jnp.where(lanes == lane_base + i, mx, tv)
                ti = jnp.where(lanes == lane_base + i, mix, ti)
                return tv, ti, mx
            return lax.fori_loop(0, K, round_fn, (tv, ti, ninf))

        tvpack, tipack, v8 = lax.cond(nv == 1, single_rounds, multi_rounds,
                                      (tvpack, tipack))
        if r % 2 == 1:
            pack_f[pl.ds((r // 2) * L, L)] = tvpack + kofs
            pack_i[pl.ds((r // 2) * L, L)] = tipack
            tvpack = ninf_v
            tipack = zero_i

        neg_h[r].wait()
        if r >= 1:
            def drain(_j, _c):
                pltpu.make_async_copy(
                    scores_hbm.at[row_base].at[pl.ds(0, GW)],
                    pstage_v.at[pl.ds(0, GW)], sscat).wait()
                return 0
            lax.fori_loop(0, q_hist[r - 1], drain, 0)

        def patch_g8(g8, q, buf=buf, row=row, v8=v8):
            pb = cmax_v[pl.ds(g8 * L, L)] >= v8
            pcnt = jnp.sum(pb.astype(jnp.int32))

            def onep(_i, carry):
                pb, q = carry
                gl = jnp.max(plsc.all_reduce_ffs(pb))
                g = g8 * L + gl
                q_c = jnp.minimum(q, QCAP - 1)
                for j in range(GROUP):
                    v = buf[pl.ds((g * GROUP + j) * L, L)]
                    pstage_v[pl.ds(q_c * GW + j * L, L)] = jnp.where(
                        v >= v8, v + kofs, neg_vec)

                @pl.when(q < QCAP)
                def _():
                    pltpu.async_copy(
                        pstage_v.at[pl.ds(q_c * GW, GW)],
                        masked_hbm.at[row].at[pl.ds(g * GW, GW)], sscat)
                pb = jnp.where(lanes == gl, False, pb)
                return pb, q + 1
            _pb, q = lax.fori_loop(0, pcnt, onep, (pb, q))
            return q
        q = lax.fori_loop(0, NG8, patch_g8, jnp.int32(0))
        q_hist[r] = jnp.minimum(q, QCAP)

    def drain_last(_j, _c):
        pltpu.make_async_copy(
            scores_hbm.at[row_base].at[pl.ds(0, GW)],
            pstage_v.at[pl.ds(0, GW)], sscat).wait()
        return 0
    lax.fori_loop(0, q_hist[RPW - 1], drain_last, 0)

    pltpu.async_copy(pack_f, vals_hbm.at[pl.ds(row_base * K, RPW * K)],
                     spack).wait()
    pltpu.async_copy(pack_i, idx_hbm.at[pl.ds(row_base * K, RPW * K)],
                     spack).wait()


def kernel(scores, k):
    kofs = jnp.full((L,), 1.0, jnp.float32) * (
        jnp.asarray(k, jnp.int32) - K).astype(jnp.float32)
    mesh = plsc.VectorSubcoreMesh(core_axis_name="c", subcore_axis_name="s",
                                  num_cores=NC, num_subcores=NS)
    f = pl.kernel(
        _body,
        out_type=[
            jax.ShapeDtypeStruct((ROWS, COLS), jnp.float32),
            jax.ShapeDtypeStruct((ROWS * K,), jnp.float32),
            jax.ShapeDtypeStruct((ROWS * K,), jnp.int32),
        ],
        mesh=mesh,
        compiler_params=pltpu.CompilerParams(needs_layout_passes=False),
        scratch_types=[
            pltpu.VMEM((COLS,), jnp.float32),
            pltpu.VMEM((COLS,), jnp.float32),
            pltpu.VMEM_SHARED((COLS,), jnp.float32),
            pltpu.VMEM((NGRP * L,), jnp.float32),
            pltpu.VMEM((NGRP,), jnp.float32),
            pltpu.VMEM((CAP + L,), jnp.float32),
            pltpu.VMEM((CAP + L,), jnp.int32),
            pltpu.VMEM((QCAP * GW,), jnp.float32),
            pltpu.VMEM((L,), jnp.float32),
            pltpu.VMEM((RPW * K,), jnp.float32),
            pltpu.VMEM((RPW * K,), jnp.int32),
            pltpu.SemaphoreType.DMA,
            pltpu.SemaphoreType.DMA,
            pltpu.SemaphoreType.DMA,
            pltpu.SemaphoreType.DMA,
            pltpu.SemaphoreType.DMA,
            pltpu.SemaphoreType.DMA,
            pltpu.SemaphoreType.DMA,
            pltpu.SemaphoreType.DMA,
        ],
    )
    masked, vals, idx = f(scores, kofs)
    return masked, vals.reshape(ROWS, K), idx.reshape(ROWS, K)


# speedup vs baseline: 1.2144x; 1.2020x over previous
import jax
import jax.numpy as jnp
from jax import lax
from jax.experimental import pallas as pl
from jax.experimental.pallas import tpu as pltpu
from jax.experimental.pallas import tpu_sc as plsc

NC, NS, L = 2, 16, 16
NW = NC * NS
ROWS, COLS = 128, 32768
RPW = ROWS // NW
K = 8
GROUP = 16
GW = GROUP * L
NGRP = COLS // GW
NG8 = NGRP // L
CAP = 2048
QCAP = 32
NEG = -100000.0
IMAX = 2**31 - 1


def _body(scores_hbm, kofs_hbm, masked_hbm, vals_hbm, idx_hbm,
          row0_v, row1_v, neg_sh, gmax_v, cmax_v, cmax2_v,
          pstage_v, kofs_v, pack_f, pack_i,
          sin0, sin1, sneg0, sneg1, sneg2, sneg3, sscat, spack):
    rowbufs = [row0_v, row1_v]
    sneg = [sneg0, sneg1, sneg2, sneg3]
    wid = lax.axis_index("s") * NC + lax.axis_index("c")
    lanes = lax.broadcasted_iota(jnp.int32, (L,), 0)
    ninf = jnp.float32(-jnp.inf)
    ninf_v = jnp.full((L,), ninf, jnp.float32)
    neg_vec = jnp.full((L,), NEG, jnp.float32)
    zero_i = jnp.zeros((L,), jnp.int32)
    row_base = wid * RPW

    in_h = [None] * RPW
    in_h[0] = pltpu.async_copy(scores_hbm.at[row_base], rowbufs[0], sin0)
    in_h_sem = [sin0, sin1]

    pltpu.sync_copy(kofs_hbm, kofs_v)
    kofs = jnp.max(kofs_v[...])

    sid = lax.axis_index("s")
    chunk = COLS // NS

    def negfill(j, _c):
        for u in range(8):
            pstage_v[pl.ds((j * 8 + u) * L, L)] = neg_vec
        return 0
    lax.fori_loop(0, chunk // (8 * L), negfill, 0)
    pltpu.sync_copy(pstage_v.at[pl.ds(0, chunk)],
                    neg_sh.at[pl.ds(sid * chunk, chunk)])
    plsc.subcore_barrier()
    neg_h = [pltpu.async_copy(neg_sh, masked_hbm.at[row_base + r], sneg[r])
             for r in range(RPW)]

    tvpack = ninf_v
    tipack = zero_i
    q_hist = [None] * RPW

    for r in range(RPW):
        buf = rowbufs[r % 2]
        row = row_base + r
        in_h[r].wait()
        if r + 1 < RPW:
            in_h[r + 1] = pltpu.async_copy(
                scores_hbm.at[row + 1], rowbufs[(r + 1) % 2],
                in_h_sem[(r + 1) % 2])

        def grp1(g, _c, buf=buf):
            acc = [ninf_v] * 4
            for j in range(GROUP):
                acc[j % 4] = jnp.maximum(
                    acc[j % 4], buf[pl.ds((g * GROUP + j) * L, L)])
            gm = jnp.maximum(jnp.maximum(acc[0], acc[1]),
                             jnp.maximum(acc[2], acc[3]))
            gmax_v[pl.ds(g * L, L)] = gm
            return 0
        lax.fori_loop(0, NGRP, grp1, 0)

        gcol = lanes * GROUP

        def qscan(g8, _c):
            acc = [ninf_v] * 4
            for c in range(GROUP):
                acc[c % 4] = jnp.maximum(
                    acc[c % 4],
                    plsc.load_gather(gmax_v, [g8 * (L * GROUP) + gcol + c]))
            cm = jnp.maximum(jnp.maximum(acc[0], acc[1]),
                             jnp.maximum(acc[2], acc[3]))
            cmax_v[pl.ds(g8 * L, L)] = cm
            return 0
        lax.fori_loop(0, NG8, qscan, 0)

        for g8 in range(NG8):
            cmax2_v[pl.ds(g8 * L, L)] = cmax_v[pl.ds(g8 * L, L)]

        lane_base = (r % 2) * K

        def round_fn(i, carry, buf=buf, lane_base=lane_base):
            tv, ti, _v8 = carry
            acc0 = jnp.maximum(cmax_v[pl.ds(0 * L, L)], cmax_v[pl.ds(1 * L, L)])
            acc1 = jnp.maximum(cmax_v[pl.ds(2 * L, L)], cmax_v[pl.ds(3 * L, L)])
            acc2 = jnp.maximum(cmax_v[pl.ds(4 * L, L)], cmax_v[pl.ds(5 * L, L)])
            acc3 = jnp.maximum(cmax_v[pl.ds(6 * L, L)], cmax_v[pl.ds(7 * L, L)])
            mx = jnp.max(jnp.maximum(jnp.maximum(acc0, acc1),
                                     jnp.maximum(acc2, acc3)))
            gsel = jnp.full((L,), IMAX, jnp.int32)
            for g8 in range(NG8):
                gsel = jnp.minimum(
                    gsel, jnp.where(cmax_v[pl.ds(g8 * L, L)] == mx,
                                    g8 * L + lanes, IMAX))
            g = jnp.min(gsel)

            def scang(j4, carry2):
                mi, sm, cnt = carry2
                for u in range(4):
                    j = j4 * 4 + u
                    base = (g * GROUP + j) * L
                    v = buf[pl.ds(base, L)]
                    ismx = v == mx
                    mi = jnp.minimum(mi, jnp.where(ismx, lanes + base, IMAX))
                    sm = jnp.maximum(sm, jnp.where(ismx, ninf_v, v))
                    cnt = cnt + ismx.astype(jnp.int32)
                return mi, sm, cnt
            mi, sm, cnt = lax.fori_loop(
                0, GROUP // 4, scang,
                (jnp.full((L,), IMAX, jnp.int32), ninf_v, zero_i))
            mix = jnp.min(mi)
            nmx = jnp.max(sm)
            ndup = jnp.sum(cnt)
            newgsm = jnp.where(ndup > 1, mx, nmx)
            plsc.store_scatter(buf, [mix + zero_i], ninf_v, mask=lanes == 0)
            plsc.store_scatter(cmax_v, [g + zero_i],
                               newgsm + jnp.zeros((L,), jnp.float32),
                               mask=lanes == 0)
            tv = jnp.where(lanes == lane_base + i, mx, tv)
            ti = jnp.where(lanes == lane_base + i, mix, ti)
            return tv, ti, mx
        tvpack, tipack, v8 = lax.fori_loop(
            0, K, round_fn, (tvpack, tipack, ninf))
        rmask = (lanes >= lane_base) & (lanes < lane_base + K)
        plsc.store_scatter(buf, [tipack], tvpack, mask=rmask)
        if r % 2 == 1:
            pack_f[pl.ds((r // 2) * L, L)] = tvpack + kofs
            pack_i[pl.ds((r // 2) * L, L)] = tipack
            tvpack = ninf_v
            tipack = zero_i

        neg_h[r].wait()
        if r >= 1:
            def drain(_j, _c):
                pltpu.make_async_copy(
                    scores_hbm.at[row_base].at[pl.ds(0, GW)],
                    pstage_v.at[pl.ds(0, GW)], sscat).wait()
                return 0
            lax.fori_loop(0, q_hist[r - 1], drain, 0)

        def patch_g8(g8, q, buf=buf, row=row, v8=v8):
            pb = cmax2_v[pl.ds(g8 * L, L)] >= v8
            pcnt = jnp.sum(pb.astype(jnp.int32))

            def onep(_i, carry):
                pb, q = carry
                gl = jnp.max(plsc.all_reduce_ffs(pb))
                g = g8 * L + gl
                q_c = jnp.minimum(q, QCAP - 1)
                for j in range(GROUP):
                    v = buf[pl.ds((g * GROUP + j) * L, L)]
                    pstage_v[pl.ds(q_c * GW + j * L, L)] = jnp.where(
                        v >= v8, v + kofs, neg_vec)

                @pl.when(q < QCAP)
                def _():
                    pltpu.async_copy(
                        pstage_v.at[pl.ds(q_c * GW, GW)],
                        masked_hbm.at[row].at[pl.ds(g * GW, GW)], sscat)
                pb = jnp.where(lanes == gl, False, pb)
                return pb, q + 1
            _pb, q = lax.fori_loop(0, pcnt, onep, (pb, q))
            return q
        q = lax.fori_loop(0, NG8, patch_g8, jnp.int32(0))
        q_hist[r] = jnp.minimum(q, QCAP)

    def drain_last(_j, _c):
        pltpu.make_async_copy(
            scores_hbm.at[row_base].at[pl.ds(0, GW)],
            pstage_v.at[pl.ds(0, GW)], sscat).wait()
        return 0
    lax.fori_loop(0, q_hist[RPW - 1], drain_last, 0)

    pltpu.async_copy(pack_f, vals_hbm.at[pl.ds(row_base * K, RPW * K)],
                     spack).wait()
    pltpu.async_copy(pack_i, idx_hbm.at[pl.ds(row_base * K, RPW * K)],
                     spack).wait()


def kernel(scores, k):
    kofs = jnp.full((L,), 1.0, jnp.float32) * (
        jnp.asarray(k, jnp.int32) - K).astype(jnp.float32)
    mesh = plsc.VectorSubcoreMesh(core_axis_name="c", subcore_axis_name="s",
                                  num_cores=NC, num_subcores=NS)
    f = pl.kernel(
        _body,
        out_type=[
            jax.ShapeDtypeStruct((ROWS, COLS), jnp.float32),
            jax.ShapeDtypeStruct((ROWS * K,), jnp.float32),
            jax.ShapeDtypeStruct((ROWS * K,), jnp.int32),
        ],
        mesh=mesh,
        compiler_params=pltpu.CompilerParams(needs_layout_passes=False),
        scratch_types=[
            pltpu.VMEM((COLS,), jnp.float32),
            pltpu.VMEM((COLS,), jnp.float32),
            pltpu.VMEM_SHARED((COLS,), jnp.float32),
            pltpu.VMEM((NGRP * L,), jnp.float32),
            pltpu.VMEM((NGRP,), jnp.float32),
            pltpu.VMEM((NGRP,), jnp.float32),
            pltpu.VMEM((QCAP * GW,), jnp.float32),
            pltpu.VMEM((L,), jnp.float32),
            pltpu.VMEM((RPW * K,), jnp.float32),
            pltpu.VMEM((RPW * K,), jnp.int32),
            pltpu.SemaphoreType.DMA,
            pltpu.SemaphoreType.DMA,
            pltpu.SemaphoreType.DMA,
            pltpu.SemaphoreType.DMA,
            pltpu.SemaphoreType.DMA,
            pltpu.SemaphoreType.DMA,
            pltpu.SemaphoreType.DMA,
            pltpu.SemaphoreType.DMA,
        ],
    )
    masked, vals, idx = f(scores, kofs)
    return masked, vals.reshape(ROWS, K), idx.reshape(ROWS, K)
